# Initial kernel scaffold; baseline (speedup 1.0000x reference)
import jax
import jax.numpy as jnp
from jax.experimental import pallas as pl


def kernel(concept_ids, edge_index, edge_attr, concept_embedding, relation_embedding, W_msg, b_msg, W_self, W_upd, b_upd, W_lin, b_lin):
    raise NotImplementedError("write your pallas kernel here")


# speedup vs baseline: 3.9829x; 3.9829x over previous
import functools

import jax
import jax.numpy as jnp
from jax import lax
from jax.experimental import pallas as pl
from jax.experimental.pallas import tpu as pltpu
from jax.experimental.pallas import tpu_sc as plsc

D = 128
NN = 10000
NE = 320000
NREL = 38
NRELP = 40
NP = 10240
NC, NS = 2, 16
NW = NC * NS
CH = 80
EPT = NE // NW
NCH = EPT // CH
APT = NP // NW

_mesh = plsc.VectorSubcoreMesh(
    core_axis_name="c", subcore_axis_name="s", num_cores=NC, num_subcores=NS)


def _wid():
    return lax.axis_index("s") * NC + lax.axis_index("c")


def _gather_x_body(ce_hbm, cid_hbm, x_hbm, idx_v, rows_v, sem):
    base = _wid() * APT

    def step(k, carry):
        off = base + k * CH
        pltpu.sync_copy(cid_hbm.at[pl.ds(off, CH)], idx_v)
        pltpu.async_copy(ce_hbm.at[idx_v], rows_v, sem).wait()
        pltpu.sync_copy(rows_v, x_hbm.at[pl.ds(off, CH)])
        return carry

    lax.fori_loop(0, APT // CH, step, 0)


_gather_x = pl.kernel(
    _gather_x_body,
    out_type=jax.ShapeDtypeStruct((NP, D), jnp.float32),
    mesh=_mesh,
    compiler_params=pltpu.CompilerParams(needs_layout_passes=False),
    scratch_types=[
        pltpu.VMEM((CH,), jnp.int32),
        pltpu.VMEM((CH, D), jnp.float32),
        pltpu.SemaphoreType.DMA,
    ],
)


def _node_pre_body(x_ref, wm_ref, bm_ref, ws_ref, rp_ref, wl_ref,
                   xm_ref, xs_ref, relcat_ref):
    x = x_ref[...]
    xm_ref[...] = jnp.dot(x, wm_ref[0:D, :],
                          preferred_element_type=jnp.float32) + bm_ref[...]
    xs_ref[...] = jnp.dot(x, ws_ref[...], preferred_element_type=jnp.float32)
    rp = rp_ref[...]
    rm = jnp.dot(rp, wm_ref[D:2 * D, :], preferred_element_type=jnp.float32)
    rl = jnp.dot(rp, wl_ref[D:2 * D, :], preferred_element_type=jnp.float32)
    relcat_ref[...] = jnp.concatenate([rm, rl], axis=1)


_node_pre = pl.pallas_call(
    _node_pre_body,
    out_shape=[
        jax.ShapeDtypeStruct((NP, D), jnp.float32),
        jax.ShapeDtypeStruct((NP, D), jnp.float32),
        jax.ShapeDtypeStruct((NRELP, 2 * D), jnp.float32),
    ],
)


EB = 2000


def _edge_bias_body(attr_ref, relcat_ref, wrm_ref, wrl_ref):
    attr = attr_ref[...]
    reli = attr[:, 0:1].astype(jnp.int32)
    w = attr[:, 1:2]
    io = lax.broadcasted_iota(jnp.int32, (EB, NRELP), 1)
    ohw = jnp.where(reli == io, w, 0.0)
    big = jnp.dot(ohw, relcat_ref[...], preferred_element_type=jnp.float32)
    wrm_ref[...] = big[:, 0:D]
    wrl_ref[...] = big[:, D:2 * D]


_edge_bias = pl.pallas_call(
    _edge_bias_body,
    grid=(NE // EB,),
    in_specs=[
        pl.BlockSpec((EB, 2), lambda i: (i, 0)),
        pl.BlockSpec((NRELP, 2 * D), lambda i: (0, 0)),
    ],
    out_specs=[
        pl.BlockSpec((EB, D), lambda i: (i, 0)),
        pl.BlockSpec((EB, D), lambda i: (i, 0)),
    ],
    out_shape=[
        jax.ShapeDtypeStruct((NE, D), jnp.float32),
        jax.ShapeDtypeStruct((NE, D), jnp.float32),
    ],
)


def _msg_agg_body(xm_hbm, wrm_hbm, src_hbm, dst_hbm, agg_hbm,
                  idx_s, idx_d, rows_v, wrm_v, shared, sem):
    c = lax.axis_index("c")
    s = lax.axis_index("s")
    wid = s * NC + c
    nstripe = NP // NS

    zero = jnp.zeros((16,), jnp.float32)
    for r in range(40):
        for j in range(8):
            rows_v[r, pl.ds(16 * j, 16)] = zero

    def zstep(i, carry):
        pltpu.sync_copy(rows_v.at[pl.ds(0, 40)],
                        shared.at[pl.ds(s * nstripe + i * 40, 40)])
        return carry

    lax.fori_loop(0, nstripe // 40, zstep, 0)
    plsc.subcore_barrier()

    def step(k, carry):
        off = wid * EPT + k * CH
        pltpu.sync_copy(src_hbm.at[pl.ds(off, CH)], idx_s)
        pltpu.sync_copy(dst_hbm.at[pl.ds(off, CH)], idx_d)
        cp = pltpu.async_copy(xm_hbm.at[idx_s], rows_v, sem)
        pltpu.sync_copy(wrm_hbm.at[pl.ds(off, CH)], wrm_v)
        cp.wait()

        def rstep(r, c2):
            for j in range(8):
                sl = pl.ds(16 * j, 16)
                rows_v[r, sl] = jnp.maximum(rows_v[r, sl] + wrm_v[r, sl], 0.0)
            return c2

        lax.fori_loop(0, CH, rstep, 0)
        pltpu.sync_copy(rows_v, shared.at[idx_d], add=True)
        return carry

    lax.fori_loop(0, NCH, step, 0)
    plsc.subcore_barrier()
    pltpu.sync_copy(shared.at[pl.ds(s * nstripe, nstripe)],
                    agg_hbm.at[c, pl.ds(s * nstripe, nstripe)])


_msg_agg = pl.kernel(
    _msg_agg_body,
    out_type=jax.ShapeDtypeStruct((NC, NP, D), jnp.float32),
    mesh=_mesh,
    compiler_params=pltpu.CompilerParams(needs_layout_passes=False),
    scratch_types=[
        pltpu.VMEM((CH,), jnp.int32),
        pltpu.VMEM((CH,), jnp.int32),
        pltpu.VMEM((CH, D), jnp.float32),
        pltpu.VMEM((CH, D), jnp.float32),
        pltpu.VMEM_SHARED((NP, D), jnp.float32),
        pltpu.SemaphoreType.DMA,
    ],
)


def _node_upd_body(agg_ref, xs_ref, wu_ref, bu_ref, wl_ref, bl_ref,
                   ls_ref, ld_ref):
    aggs = agg_ref[0, 0:NN, :] + agg_ref[1, 0:NN, :]
    x2 = jnp.maximum(
        jnp.dot(aggs, wu_ref[...], preferred_element_type=jnp.float32)
        + xs_ref[0:NN, :] + bu_ref[...], 0.0)
    ls_ref[...] = jnp.dot(x2, wl_ref[0:D, :],
                          preferred_element_type=jnp.float32) + bl_ref[...]
    ld_ref[...] = jnp.dot(x2, wl_ref[2 * D:3 * D, :],
                          preferred_element_type=jnp.float32)


_node_upd = pl.pallas_call(
    _node_upd_body,
    out_shape=[
        jax.ShapeDtypeStruct((NN, D), jnp.float32),
        jax.ShapeDtypeStruct((NN, D), jnp.float32),
    ],
)


def _edge_out_body(ls_hbm, ld_hbm, wrl_hbm, src_hbm, dst_hbm, attr_hbm,
                   cid_hbm, enc_hbm, tid_hbm,
                   idx_s, idx_d, a_v, b_v, c_v, cid_v, attr_v, tid_v,
                   sem, sem2):
    wid = _wid()
    pltpu.sync_copy(cid_hbm, cid_v)
    lanes = lax.iota(jnp.int32, 16)
    col0 = jnp.zeros((16,), jnp.int32)
    col1 = jnp.full((16,), 1, jnp.int32)
    col2 = jnp.full((16,), 2, jnp.int32)

    def step(k, carry):
        off = wid * EPT + k * CH
        pltpu.sync_copy(src_hbm.at[pl.ds(off, CH)], idx_s)
        pltpu.sync_copy(dst_hbm.at[pl.ds(off, CH)], idx_d)
        cp1 = pltpu.async_copy(ls_hbm.at[idx_s], a_v, sem)
        cp2 = pltpu.async_copy(ld_hbm.at[idx_d], b_v, sem2)
        pltpu.sync_copy(wrl_hbm.at[pl.ds(off, CH)], c_v)
        pltpu.sync_copy(attr_hbm.at[pl.ds(off, CH)], attr_v)
        cp1.wait()
        cp2.wait()

        def rstep(r, c2):
            for j in range(8):
                sl = pl.ds(16 * j, 16)
                a_v[r, sl] = a_v[r, sl] + b_v[r, sl] + c_v[r, sl]
            return c2

        lax.fori_loop(0, CH, rstep, 0)

        for i in range(CH // 16):
            rows16 = jnp.full((16,), i * 16, jnp.int32) + lanes
            sv = idx_s[pl.ds(i * 16, 16)]
            dv = idx_d[pl.ds(i * 16, 16)]
            cs = plsc.load_gather(cid_v, [sv])
            cd = plsc.load_gather(cid_v, [dv])
            relf = plsc.load_gather(attr_v, [rows16, col0])
            ri = relf.astype(jnp.int32)
            plsc.store_scatter(tid_v, [rows16, col0], cs)
            plsc.store_scatter(tid_v, [rows16, col1], ri)
            plsc.store_scatter(tid_v, [rows16, col2], cd)

        pltpu.sync_copy(a_v, enc_hbm.at[pl.ds(off, CH)])
        pltpu.sync_copy(tid_v, tid_hbm.at[pl.ds(off, CH)])
        return carry

    lax.fori_loop(0, NCH, step, 0)


_edge_out = pl.kernel(
    _edge_out_body,
    out_type=[
        jax.ShapeDtypeStruct((NE, D), jnp.float32),
        jax.ShapeDtypeStruct((NE, 3), jnp.int32),
    ],
    mesh=_mesh,
    compiler_params=pltpu.CompilerParams(needs_layout_passes=False),
    scratch_types=[
        pltpu.VMEM((CH,), jnp.int32),
        pltpu.VMEM((CH,), jnp.int32),
        pltpu.VMEM((CH, D), jnp.float32),
        pltpu.VMEM((CH, D), jnp.float32),
        pltpu.VMEM((CH, D), jnp.float32),
        pltpu.VMEM((NN,), jnp.int32),
        pltpu.VMEM((CH, 2), jnp.float32),
        pltpu.VMEM((CH, 3), jnp.int32),
        pltpu.SemaphoreType.DMA,
        pltpu.SemaphoreType.DMA,
    ],
)


def kernel(concept_ids, edge_index, edge_attr, concept_embedding,
           relation_embedding, W_msg, b_msg, W_self, W_upd, b_upd,
           W_lin, b_lin):
    src = edge_index[0]
    dst = edge_index[1]
    cid_pad = jnp.concatenate(
        [concept_ids, jnp.zeros((NP - NN,), jnp.int32)])
    rp = jnp.pad(relation_embedding, ((0, NRELP - NREL), (0, 0)))

    x = _gather_x(concept_embedding, cid_pad)
    xm, xs, relcat = _node_pre(x, W_msg, b_msg, W_self, rp, W_lin)
    wrm, wrl = _edge_bias(edge_attr, relcat)
    agg2 = _msg_agg(xm, wrm, src, dst)
    ls, ld = _node_upd(agg2, xs, W_upd, b_upd, W_lin, b_lin)
    enc, tid = _edge_out(ls, ld, wrl, src, dst, edge_attr, concept_ids)
    return enc, tid
